# Initial kernel scaffold; baseline (speedup 1.0000x reference)
import jax
import jax.numpy as jnp
from jax.experimental import pallas as pl


def kernel(psi_final_abstract, quantum_representations, top_k):
    raise NotImplementedError("write your pallas kernel here")


# speedup vs baseline: 1.1977x; 1.1977x over previous
import functools

import jax
import jax.numpy as jnp
from jax import lax
from jax.experimental import pallas as pl
from jax.experimental.pallas import tpu as pltpu
from jax.experimental.pallas import tpu_sc as plsc

V = 100000
D = 128
K = 5

NC = 2
NS = 16
NW = NC * NS
PER_W = 3136
PAD_V = NW * PER_W
GROUPS = PER_W // 16

BLK = 784
GRID = PAD_V // BLK

_NEG = float("-inf")
_IMAX = 2**31 - 1


def _dss_body(psi_ref, tab_ref, d_ref, ss_ref):
    psi = psi_ref[:, :]
    nrm = jnp.sqrt(jnp.sum(psi * psi)) + jnp.float32(1e-9)
    pn = psi / nrm
    tab = tab_ref[:, :]
    d_ref[:, :] = lax.dot_general(tab, pn, (((1,), (1,)), ((), ())),
                                  precision=lax.Precision.HIGHEST)
    ss_ref[:, :] = jnp.sum(tab * tab, axis=1, keepdims=True)


def _dss(psi2d, table):
    return pl.pallas_call(
        _dss_body,
        grid=(GRID,),
        in_specs=[
            pl.BlockSpec((1, D), lambda i: (0, 0)),
            pl.BlockSpec((BLK, D), lambda i: (i, 0)),
        ],
        out_specs=[
            pl.BlockSpec((BLK, 1), lambda i: (i, 0)),
            pl.BlockSpec((BLK, 1), lambda i: (i, 0)),
        ],
        out_shape=[
            jax.ShapeDtypeStruct((PAD_V, 1), jnp.float32),
            jax.ShapeDtypeStruct((PAD_V, 1), jnp.float32),
        ],
    )(psi2d, table)


@functools.partial(
    pl.kernel,
    mesh=plsc.VectorSubcoreMesh(core_axis_name="c", subcore_axis_name="s"),
    out_type=[
        jax.ShapeDtypeStruct((NW, 16), jnp.float32),
        jax.ShapeDtypeStruct((NW, 16), jnp.int32),
    ],
    scratch_types=[
        pltpu.VMEM((PER_W,), jnp.float32),
        pltpu.VMEM((PER_W,), jnp.float32),
        pltpu.VMEM((PER_W,), jnp.float32),
        pltpu.VMEM((16,), jnp.float32),
        pltpu.VMEM((16,), jnp.int32),
    ],
    compiler_params=pltpu.CompilerParams(needs_layout_passes=False),
)
def _topk_sc(d_hbm, ss_hbm, outv_hbm, outi_hbm, bufd, bufss, sims, candv,
             candi):
    c = lax.axis_index("c")
    s = lax.axis_index("s")
    wid = s * NC + c
    base = wid * PER_W
    pltpu.sync_copy(d_hbm.at[pl.ds(base, PER_W)], bufd)
    pltpu.sync_copy(ss_hbm.at[pl.ds(base, PER_W)], bufss)
    iota = lax.iota(jnp.int32, 16)
    lane0 = iota == 0
    vlim = jnp.int32(V) - base

    def simbody(g, _):
        d = bufd[pl.ds(g * 16, 16)]
        ss = bufss[pl.ds(g * 16, 16)]
        y = plsc.bitcast(
            jnp.int32(0x5F3759DF) - (plsc.bitcast(ss, jnp.int32) >> 1),
            jnp.float32)
        half_ss = ss * jnp.float32(0.5)
        for _ in range(3):
            y = y * (jnp.float32(1.5) - half_ss * y * y)
        sim = d / (ss * y + jnp.float32(1e-9))
        valid = (iota + g * 16) < vlim
        sims[pl.ds(g * 16, 16)] = jnp.where(valid, sim, _NEG)
        return 0

    lax.fori_loop(0, GROUPS, simbody, 0, unroll=4)

    cv = jnp.full((16,), _NEG, jnp.float32)
    ci = jnp.zeros((16,), jnp.int32)
    for k in range(K):
        def body(g, carry):
            bv, bi = carry
            v = sims[pl.ds(g * 16, 16)]
            m = v > bv
            bv = jnp.where(m, v, bv)
            bi = jnp.where(m, iota + g * 16, bi)
            return bv, bi
        bv, bi = lax.fori_loop(
            0, GROUPS, body,
            (jnp.full((16,), _NEG, jnp.float32), jnp.zeros((16,), jnp.int32)),
            unroll=4)
        wv = jnp.max(bv)
        wi = jnp.min(jnp.where(bv == wv, bi, _IMAX))
        cv = jnp.where(iota == k, wv, cv)
        ci = jnp.where(iota == k, wi + base, ci)
        if k < K - 1:
            plsc.store_scatter(sims, [jnp.full((16,), wi, jnp.int32)],
                               jnp.full((16,), _NEG, jnp.float32), mask=lane0)
    candv[...] = cv
    candi[...] = ci
    pltpu.sync_copy(candv, outv_hbm.at[wid])
    pltpu.sync_copy(candi, outi_hbm.at[wid])


def _merge_body(val_ref, idx_ref, outv_ref, outi_ref):
    vals = val_ref[:, :]
    idxs = idx_ref[:, :]
    resv = jnp.full((8, 1), _NEG, jnp.float32)
    resi = jnp.zeros((8, 1), jnp.int32)
    rows8 = lax.broadcasted_iota(jnp.int32, (8, 1), 0)
    for k in range(K):
        m = jnp.max(vals)
        eq = vals == m
        wi = jnp.min(jnp.where(eq, idxs, _IMAX))
        vals = jnp.where(eq & (idxs == wi), _NEG, vals)
        resv = jnp.where(rows8 == k, m, resv)
        resi = jnp.where(rows8 == k, wi, resi)
    outv_ref[:, :] = resv
    outi_ref[:, :] = resi


def _merge(cv, ci):
    return pl.pallas_call(
        _merge_body,
        out_shape=[
            jax.ShapeDtypeStruct((8, 1), jnp.float32),
            jax.ShapeDtypeStruct((8, 1), jnp.int32),
        ],
    )(cv, ci)


def kernel(psi_final_abstract, quantum_representations, top_k):
    del top_k
    psi2d = psi_final_abstract.reshape(1, D)
    d, ss = _dss(psi2d, quantum_representations)
    cv, ci = _topk_sc(d.reshape(PAD_V), ss.reshape(PAD_V))
    tv, ti = _merge(cv, ci)
    return tv[:K, 0], ti[:K, 0]


# speedup vs baseline: 1.7814x; 1.4873x over previous
import functools

import jax
import jax.numpy as jnp
from jax import lax
from jax.experimental import pallas as pl
from jax.experimental.pallas import tpu as pltpu
from jax.experimental.pallas import tpu_sc as plsc

V = 100000
D = 128
K = 5

NC = 2
NS = 16
NW = NC * NS
PER_W = 3136
PAD_V = NW * PER_W
GROUPS = PER_W // 16

BLK = 1024
CHUNKS = BLK // D
GRID = PAD_V // BLK
ROWS_OUT = PAD_V // D

_NEG = float("-inf")
_IMAX = 2**31 - 1


def _dss_body(psi_ref, tab_ref, d_ref, ss_ref):
    psi = psi_ref[:, :]
    nrm = jnp.sqrt(jnp.sum(psi * psi)) + jnp.float32(1e-9)
    pn = psi / nrm
    one = jnp.ones((1, D), jnp.float32)
    ds, sss = [], []
    for c in range(CHUNKS):
        chunk = tab_ref[c * D:(c + 1) * D, :]
        dc = jnp.sum(chunk * pn, axis=1, keepdims=True)
        sc = jnp.sum(chunk * chunk, axis=1, keepdims=True)
        ds.append(lax.transpose(dc, (1, 0)))
        sss.append(lax.transpose(sc, (1, 0)))
    d_ref[:, :] = jnp.concatenate(ds, axis=0)
    ss_ref[:, :] = jnp.concatenate(sss, axis=0)


def _dss(psi2d, table):
    return pl.pallas_call(
        _dss_body,
        grid=(GRID,),
        in_specs=[
            pl.BlockSpec((1, D), lambda i: (0, 0)),
            pl.BlockSpec((BLK, D), lambda i: (i, 0)),
        ],
        out_specs=[
            pl.BlockSpec((CHUNKS, D), lambda i: (i, 0)),
            pl.BlockSpec((CHUNKS, D), lambda i: (i, 0)),
        ],
        out_shape=[
            jax.ShapeDtypeStruct((ROWS_OUT, D), jnp.float32),
            jax.ShapeDtypeStruct((ROWS_OUT, D), jnp.float32),
        ],
    )(psi2d, table)


@functools.partial(
    pl.kernel,
    mesh=plsc.VectorSubcoreMesh(core_axis_name="c", subcore_axis_name="s"),
    out_type=[
        jax.ShapeDtypeStruct((NW, 16), jnp.float32),
        jax.ShapeDtypeStruct((NW, 16), jnp.int32),
    ],
    scratch_types=[
        pltpu.VMEM((PER_W,), jnp.float32),
        pltpu.VMEM((PER_W,), jnp.float32),
        pltpu.VMEM((PER_W,), jnp.float32),
        pltpu.VMEM((16,), jnp.float32),
        pltpu.VMEM((16,), jnp.int32),
    ],
    compiler_params=pltpu.CompilerParams(needs_layout_passes=False),
)
def _topk_sc(d_hbm, ss_hbm, outv_hbm, outi_hbm, bufd, bufss, sims, candv,
             candi):
    c = lax.axis_index("c")
    s = lax.axis_index("s")
    wid = s * NC + c
    base = wid * PER_W
    pltpu.sync_copy(d_hbm.at[pl.ds(base, PER_W)], bufd)
    pltpu.sync_copy(ss_hbm.at[pl.ds(base, PER_W)], bufss)
    iota = lax.iota(jnp.int32, 16)
    lane0 = iota == 0
    vlim = jnp.int32(V) - base

    def simbody(g, _):
        d = bufd[pl.ds(g * 16, 16)]
        ss = bufss[pl.ds(g * 16, 16)]
        y = plsc.bitcast(
            jnp.int32(0x5F3759DF) - (plsc.bitcast(ss, jnp.int32) >> 1),
            jnp.float32)
        half_ss = ss * jnp.float32(0.5)
        for _ in range(3):
            y = y * (jnp.float32(1.5) - half_ss * y * y)
        sim = d / (ss * y + jnp.float32(1e-9))
        valid = (iota + g * 16) < vlim
        sims[pl.ds(g * 16, 16)] = jnp.where(valid, sim, _NEG)
        return 0

    lax.fori_loop(0, GROUPS, simbody, 0, unroll=4)

    cv = jnp.full((16,), _NEG, jnp.float32)
    ci = jnp.zeros((16,), jnp.int32)
    for k in range(K):
        def body(g, carry):
            bv, bi = carry
            v = sims[pl.ds(g * 16, 16)]
            m = v > bv
            bv = jnp.where(m, v, bv)
            bi = jnp.where(m, iota + g * 16, bi)
            return bv, bi
        bv, bi = lax.fori_loop(
            0, GROUPS, body,
            (jnp.full((16,), _NEG, jnp.float32), jnp.zeros((16,), jnp.int32)),
            unroll=4)
        wv = jnp.max(bv)
        wi = jnp.min(jnp.where(bv == wv, bi, _IMAX))
        cv = jnp.where(iota == k, wv, cv)
        ci = jnp.where(iota == k, wi + base, ci)
        if k < K - 1:
            plsc.store_scatter(sims, [jnp.full((16,), wi, jnp.int32)],
                               jnp.full((16,), _NEG, jnp.float32), mask=lane0)
    candv[...] = cv
    candi[...] = ci
    pltpu.sync_copy(candv, outv_hbm.at[wid])
    pltpu.sync_copy(candi, outi_hbm.at[wid])


def _merge_body(val_ref, idx_ref, outv_ref, outi_ref):
    vals = val_ref[:, :]
    idxs = idx_ref[:, :]
    resv = jnp.full((8, 1), _NEG, jnp.float32)
    resi = jnp.zeros((8, 1), jnp.int32)
    rows8 = lax.broadcasted_iota(jnp.int32, (8, 1), 0)
    for k in range(K):
        m = jnp.max(vals)
        eq = vals == m
        wi = jnp.min(jnp.where(eq, idxs, _IMAX))
        vals = jnp.where(eq & (idxs == wi), _NEG, vals)
        resv = jnp.where(rows8 == k, m, resv)
        resi = jnp.where(rows8 == k, wi, resi)
    outv_ref[:, :] = resv
    outi_ref[:, :] = resi


def _merge(cv, ci):
    return pl.pallas_call(
        _merge_body,
        out_shape=[
            jax.ShapeDtypeStruct((8, 1), jnp.float32),
            jax.ShapeDtypeStruct((8, 1), jnp.int32),
        ],
    )(cv, ci)


def kernel(psi_final_abstract, quantum_representations, top_k):
    del top_k
    psi2d = psi_final_abstract.reshape(1, D)
    d, ss = _dss(psi2d, quantum_representations)
    cv, ci = _topk_sc(d.reshape(PAD_V), ss.reshape(PAD_V))
    tv, ti = _merge(cv, ci)
    return tv[:K, 0], ti[:K, 0]


# speedup vs baseline: 3.3749x; 1.8945x over previous
import functools

import jax
import jax.numpy as jnp
from jax import lax
from jax.experimental import pallas as pl
from jax.experimental.pallas import tpu as pltpu
from jax.experimental.pallas import tpu_sc as plsc

V = 100000
D = 128
K = 5

NC = 2
NS = 16
NW = NC * NS
PER_W = 3136
PAD_V = NW * PER_W
GROUPS = PER_W // 16

BLK = 14336
CHUNKS = BLK // D
GRID = PAD_V // BLK
ROWS_OUT = PAD_V // D

_NEG = float("-inf")
_IMAX = 2**31 - 1


def _dss_body(psi_ref, tab_ref, d_ref, ss_ref):
    psi = psi_ref[:, :]
    nrm = jnp.sqrt(jnp.sum(psi * psi)) + jnp.float32(1e-9)
    pn = psi / nrm
    one = jnp.ones((1, D), jnp.bfloat16)
    tdims = (((1,), (1,)), ((), ()))
    f32 = jnp.float32
    ds, sss = [], []
    for c in range(CHUNKS):
        chunk = tab_ref[c * D:(c + 1) * D, :]
        dc = jnp.sum(chunk * pn, axis=1, keepdims=True)
        ds.append(lax.transpose(dc, (1, 0)))
        sq = chunk * chunk
        s_hi = sq.astype(jnp.bfloat16)
        s_lo = (sq - s_hi.astype(f32)).astype(jnp.bfloat16)
        sss.append(lax.dot_general(one, s_hi, tdims, preferred_element_type=f32)
                   + lax.dot_general(one, s_lo, tdims,
                                     preferred_element_type=f32))
    d_ref[:, :] = jnp.concatenate(ds, axis=0)
    ss_ref[:, :] = jnp.concatenate(sss, axis=0)


def _dss(psi2d, table):
    return pl.pallas_call(
        _dss_body,
        grid=(GRID,),
        in_specs=[
            pl.BlockSpec((1, D), lambda i: (0, 0)),
            pl.BlockSpec((BLK, D), lambda i: (i, 0)),
        ],
        out_specs=[
            pl.BlockSpec((CHUNKS, D), lambda i: (i, 0)),
            pl.BlockSpec((CHUNKS, D), lambda i: (i, 0)),
        ],
        out_shape=[
            jax.ShapeDtypeStruct((ROWS_OUT, D), jnp.float32),
            jax.ShapeDtypeStruct((ROWS_OUT, D), jnp.float32),
        ],
    )(psi2d, table)


@functools.partial(
    pl.kernel,
    mesh=plsc.VectorSubcoreMesh(core_axis_name="c", subcore_axis_name="s"),
    out_type=[
        jax.ShapeDtypeStruct((NW, 16), jnp.float32),
        jax.ShapeDtypeStruct((NW, 16), jnp.int32),
    ],
    scratch_types=[
        pltpu.VMEM((PER_W,), jnp.float32),
        pltpu.VMEM((PER_W,), jnp.float32),
        pltpu.VMEM((PER_W,), jnp.float32),
        pltpu.VMEM((16,), jnp.float32),
        pltpu.VMEM((16,), jnp.int32),
    ],
    compiler_params=pltpu.CompilerParams(needs_layout_passes=False),
)
def _topk_sc(d_hbm, ss_hbm, outv_hbm, outi_hbm, bufd, bufss, sims, candv,
             candi):
    c = lax.axis_index("c")
    s = lax.axis_index("s")
    wid = s * NC + c
    base = wid * PER_W
    pltpu.sync_copy(d_hbm.at[pl.ds(base, PER_W)], bufd)
    pltpu.sync_copy(ss_hbm.at[pl.ds(base, PER_W)], bufss)
    iota = lax.iota(jnp.int32, 16)
    lane0 = iota == 0
    vlim = jnp.int32(V) - base

    def simbody(g, _):
        d = bufd[pl.ds(g * 16, 16)]
        ss = bufss[pl.ds(g * 16, 16)]
        y = plsc.bitcast(
            jnp.int32(0x5F3759DF) - (plsc.bitcast(ss, jnp.int32) >> 1),
            jnp.float32)
        half_ss = ss * jnp.float32(0.5)
        for _ in range(3):
            y = y * (jnp.float32(1.5) - half_ss * y * y)
        sim = d / (ss * y + jnp.float32(1e-9))
        valid = (iota + g * 16) < vlim
        sims[pl.ds(g * 16, 16)] = jnp.where(valid, sim, _NEG)
        return 0

    lax.fori_loop(0, GROUPS, simbody, 0, unroll=4)

    cv = jnp.full((16,), _NEG, jnp.float32)
    ci = jnp.zeros((16,), jnp.int32)
    for k in range(K):
        def body(g, carry):
            bv, bi = carry
            v = sims[pl.ds(g * 16, 16)]
            m = v > bv
            bv = jnp.where(m, v, bv)
            bi = jnp.where(m, iota + g * 16, bi)
            return bv, bi
        bv, bi = lax.fori_loop(
            0, GROUPS, body,
            (jnp.full((16,), _NEG, jnp.float32), jnp.zeros((16,), jnp.int32)),
            unroll=4)
        wv = jnp.max(bv)
        wi = jnp.min(jnp.where(bv == wv, bi, _IMAX))
        cv = jnp.where(iota == k, wv, cv)
        ci = jnp.where(iota == k, wi + base, ci)
        if k < K - 1:
            plsc.store_scatter(sims, [jnp.full((16,), wi, jnp.int32)],
                               jnp.full((16,), _NEG, jnp.float32), mask=lane0)
    candv[...] = cv
    candi[...] = ci
    pltpu.sync_copy(candv, outv_hbm.at[wid])
    pltpu.sync_copy(candi, outi_hbm.at[wid])


def _merge_body(val_ref, idx_ref, outv_ref, outi_ref):
    vals = val_ref[:, :]
    idxs = idx_ref[:, :]
    resv = jnp.full((8, 1), _NEG, jnp.float32)
    resi = jnp.zeros((8, 1), jnp.int32)
    rows8 = lax.broadcasted_iota(jnp.int32, (8, 1), 0)
    for k in range(K):
        m = jnp.max(vals)
        eq = vals == m
        wi = jnp.min(jnp.where(eq, idxs, _IMAX))
        vals = jnp.where(eq & (idxs == wi), _NEG, vals)
        resv = jnp.where(rows8 == k, m, resv)
        resi = jnp.where(rows8 == k, wi, resi)
    outv_ref[:, :] = resv
    outi_ref[:, :] = resi


def _merge(cv, ci):
    return pl.pallas_call(
        _merge_body,
        out_shape=[
            jax.ShapeDtypeStruct((8, 1), jnp.float32),
            jax.ShapeDtypeStruct((8, 1), jnp.int32),
        ],
    )(cv, ci)


def kernel(psi_final_abstract, quantum_representations, top_k):
    del top_k
    psi2d = psi_final_abstract.reshape(1, D)
    d, ss = _dss(psi2d, quantum_representations)
    cv, ci = _topk_sc(d.reshape(PAD_V), ss.reshape(PAD_V))
    tv, ti = _merge(cv, ci)
    return tv[:K, 0], ti[:K, 0]
